# baseline (speedup 1.0000x reference)
import jax
import jax.numpy as jnp
from jax import lax
from jax.experimental import pallas as pl
from jax.experimental.pallas import tpu as pltpu
from jax.experimental.pallas import tpu_sc as plsc

B = 16384
L = 50
DLAB = 32
NEG = -1e9

_info = plsc.get_sparse_core_info()
NC = _info.num_cores
NS = _info.num_subcores
NW = NC * NS
EPW = B // NW
CE = 16
NCHUNK = EPW // CE
FE = 128
NFCH = EPW // FE

_f32 = jnp.float32
_i32 = jnp.int32


def _splat_i(v):
    return jnp.full((16,), v, _i32)


NP = DLAB // 2


def _unpack_pair(g):
    return plsc.unpack(plsc.bitcast(g, jnp.bfloat16),
                       format=plsc.PackFormat.INTERLEAVED,
                       preferred_element_type=_f32)


def _pool_compute(emb_ref, scores_ref, len_ref, p, eofs, pool_ref, w_v):
    iota = lax.iota(_i32, 16)
    rowb = iota * L + eofs
    lenv = jnp.maximum(len_ref[pl.ds(p * CE, CE)], 1)
    cols = [(iota + p0) & (NP - 1) for p0 in range(NP)]
    w_e = [plsc.load_gather(w_v, [cols[p0]]) for p0 in range(NP)]
    w_o = [plsc.load_gather(w_v, [cols[p0] + NP]) for p0 in range(NP)]

    def s_body(l, m):
        row = rowb + l
        a = [jnp.zeros((16,), _f32) for _ in range(4)]
        for p0 in range(NP):
            g = plsc.load_gather(emb_ref, [row, cols[p0]])
            ge, go = _unpack_pair(g)
            a[p0 % 2] = a[p0 % 2] + ge * w_e[p0]
            a[2 + p0 % 2] = a[2 + p0 % 2] + go * w_o[p0]
        acc = (a[0] + a[1]) + (a[2] + a[3])
        s = jnp.where(l < lenv, acc, jnp.full((16,), NEG, _f32))
        scores_ref[pl.ds(l * 16, 16)] = s
        return jnp.maximum(m, s)

    m = lax.fori_loop(0, L, s_body, jnp.full((16,), NEG, _f32))

    def e_body(l, ssum):
        s = scores_ref[pl.ds(l * 16, 16)]
        e = jnp.exp(s - m)
        scores_ref[pl.ds(l * 16, 16)] = e
        return ssum + e

    ssum = lax.fori_loop(0, L, e_body, jnp.zeros((16,), _f32))
    r = 1.0 / ssum

    half_np = NP // 2
    for half in range(2):
        def w_body(l, accs):
            e = scores_ref[pl.ds(l * 16, 16)]
            row = rowb + l
            new = []
            for k in range(half_np):
                p0 = half * half_np + k
                g = plsc.load_gather(emb_ref, [row, cols[p0]])
                ge, go = _unpack_pair(g)
                new.append(accs[2 * k] + e * ge)
                new.append(accs[2 * k + 1] + e * go)
            return tuple(new)

        res = lax.fori_loop(0, L, w_body, tuple(
            jnp.zeros((16,), _f32) for _ in range(NP)))
        for k in range(half_np):
            p0 = half * half_np + k
            plsc.store_scatter(pool_ref, [iota, cols[p0]],
                               res[2 * k] * r)
            plsc.store_scatter(pool_ref, [iota, cols[p0] + NP],
                               res[2 * k + 1] * r)


def _sc_body(user_id, gender_id, job_id, user_city_id, age_bucket,
             ulab, ulen, item_id, category_id, item_city_id, ilab, ilen,
             uid_tab, gen_tab, job_tab, city_tab, age_tab,
             iid_tab, cat_tab, lab_tab, pool_w,
             uid_o, gen_o, job_o, ucity_o, age_o, upool_o,
             iid_o, cat_o, icity_o, ipool_o,
             uidx_v, iidx_v, uemb_v, iemb_v, scores_v,
             ulen_v, ilen_v, w_v, pool_u_v, pool_i_v,
             fi_uid, fi_gen, fi_job, fi_ucity, fi_age, fi_iid, fi_cat,
             fi_icity,
             fr_uid, fr_gen, fr_job, fr_ucity, fr_age, fr_iid, fr_cat,
             fr_icity,
             sem_a, sem_b0, sem_b1, sem_c):
    wid = lax.axis_index("s") * NC + lax.axis_index("c")
    wbase = pl.multiple_of(wid * EPW, 128)
    pltpu.sync_copy(pool_w, w_v)

    fields = [
        (user_id, uid_tab, fi_uid, fr_uid, uid_o),
        (gender_id, gen_tab, fi_gen, fr_gen, gen_o),
        (job_id, job_tab, fi_job, fr_job, job_o),
        (user_city_id, city_tab, fi_ucity, fr_ucity, ucity_o),
        (age_bucket, age_tab, fi_age, fr_age, age_o),
        (item_id, iid_tab, fi_iid, fr_iid, iid_o),
        (category_id, cat_tab, fi_cat, fr_cat, cat_o),
        (item_city_id, city_tab, fi_icity, fr_icity, icity_o),
    ]
    sem_b = (sem_b0, sem_b1)

    def fire_idx(nbase, p1):
        ds = [pltpu.async_copy(ulab.at[pl.ds(nbase, CE)], uidx_v.at[p1],
                               sem_a),
              pltpu.async_copy(ilab.at[pl.ds(nbase, CE)], iidx_v.at[p1],
                               sem_a),
              pltpu.async_copy(ulen.at[pl.ds(nbase, CE)],
                               ulen_v.at[pl.ds(p1 * CE, CE)], sem_a),
              pltpu.async_copy(ilen.at[pl.ds(nbase, CE)],
                               ilen_v.at[pl.ds(p1 * CE, CE)], sem_a)]
        ds += [pltpu.async_copy(src.at[pl.ds(nbase, CE)], idx_v.at[p1],
                                sem_a)
               for (src, _, idx_v, _, _) in fields]
        for d in ds:
            d.wait()

    def fire_gathers(p1):
        s = sem_b[p1]
        for j in range(CE):
            pltpu.async_copy(lab_tab.at[uidx_v.at[p1, j]],
                             uemb_v.at[pl.ds(p1 * CE * L + j * L, L)], s)
            pltpu.async_copy(lab_tab.at[iidx_v.at[p1, j]],
                             iemb_v.at[pl.ds(p1 * CE * L + j * L, L)], s)
        for (_, tab, idx_v, row_v, _) in fields:
            pltpu.async_copy(tab.at[idx_v.at[p1]], row_v.at[p1], s)

    def wait_gathers(p):
        s = sem_b[p]
        dummy = lab_tab.at[pl.ds(0, CE * L)]
        pltpu.make_async_copy(dummy, uemb_v.at[pl.ds(p * CE * L, CE * L)],
                              s).wait()
        pltpu.make_async_copy(dummy, iemb_v.at[pl.ds(p * CE * L, CE * L)],
                              s).wait()
        for (_, _, _, row_v, out) in fields:
            pltpu.make_async_copy(out.at[pl.ds(0, CE)], row_v.at[p],
                                  s).wait()

    fire_idx(wbase, 0)
    fire_gathers(0)

    def c_body(c, carry):
        even = (c & 1) == 0
        not_last = c < NCHUNK - 1
        nbase = pl.multiple_of(wbase + (c + 1) * CE, 16)
        base = pl.multiple_of(wbase + c * CE, 16)
        p = c & 1

        @pl.when(even & not_last)
        def _():
            fire_idx(nbase, 1)
            fire_gathers(1)

        @pl.when(jnp.logical_not(even) & not_last)
        def _():
            fire_idx(nbase, 0)
            fire_gathers(0)

        @pl.when(even)
        def _():
            wait_gathers(0)

        @pl.when(jnp.logical_not(even))
        def _():
            wait_gathers(1)

        eofs = p * CE * L
        _pool_compute(uemb_v, scores_v, ulen_v, p, eofs, pool_u_v, w_v)
        du = pltpu.async_copy(pool_u_v, upool_o.at[pl.ds(base, CE)], sem_c)
        _pool_compute(iemb_v, scores_v, ilen_v, p, eofs, pool_i_v, w_v)
        di = pltpu.async_copy(pool_i_v, ipool_o.at[pl.ds(base, CE)], sem_c)
        dfs = [pltpu.async_copy(row_v.at[p], out.at[pl.ds(base, CE)], sem_c)
               for (_, _, _, row_v, out) in fields]
        du.wait()
        di.wait()
        for d in dfs:
            d.wait()
        return carry

    lax.fori_loop(0, NCHUNK, c_body, 0)


def _sc_gather_pool(user_id, gender_id, job_id, user_city_id, age_bucket,
                    ulab, ulen, item_id, category_id, item_city_id,
                    ilab, ilen, uid_tab, gen_tab, job_tab, city_tab,
                    age_tab, iid_tab, cat_tab, lab_tab, pool_w):
    out_type = [
        jax.ShapeDtypeStruct((B, 64), _f32),
        jax.ShapeDtypeStruct((B, 16), _f32),
        jax.ShapeDtypeStruct((B, 16), _f32),
        jax.ShapeDtypeStruct((B, 16), _f32),
        jax.ShapeDtypeStruct((B, 16), _f32),
        jax.ShapeDtypeStruct((B, 32), _f32),
        jax.ShapeDtypeStruct((B, 64), _f32),
        jax.ShapeDtypeStruct((B, 32), _f32),
        jax.ShapeDtypeStruct((B, 16), _f32),
        jax.ShapeDtypeStruct((B, 32), _f32),
    ]
    scratch = [
        pltpu.VMEM((2, CE, L), _i32),
        pltpu.VMEM((2, CE, L), _i32),
        pltpu.VMEM((2 * CE * L, NP), _i32),
        pltpu.VMEM((2 * CE * L, NP), _i32),
        pltpu.VMEM((L * 16,), _f32),
        pltpu.VMEM((2 * CE,), _i32),
        pltpu.VMEM((2 * CE,), _i32),
        pltpu.VMEM((DLAB,), _f32),
        pltpu.VMEM((CE, DLAB), _f32),
        pltpu.VMEM((CE, DLAB), _f32),
        pltpu.VMEM((2, CE), _i32),
        pltpu.VMEM((2, CE), _i32),
        pltpu.VMEM((2, CE), _i32),
        pltpu.VMEM((2, CE), _i32),
        pltpu.VMEM((2, CE), _i32),
        pltpu.VMEM((2, CE), _i32),
        pltpu.VMEM((2, CE), _i32),
        pltpu.VMEM((2, CE), _i32),
        pltpu.VMEM((2, CE, 64), _f32),
        pltpu.VMEM((2, CE, 16), _f32),
        pltpu.VMEM((2, CE, 16), _f32),
        pltpu.VMEM((2, CE, 16), _f32),
        pltpu.VMEM((2, CE, 16), _f32),
        pltpu.VMEM((2, CE, 64), _f32),
        pltpu.VMEM((2, CE, 32), _f32),
        pltpu.VMEM((2, CE, 16), _f32),
        pltpu.SemaphoreType.DMA,
        pltpu.SemaphoreType.DMA,
        pltpu.SemaphoreType.DMA,
        pltpu.SemaphoreType.DMA,
    ]
    fn = pl.kernel(
        _sc_body,
        out_type=out_type,
        mesh=plsc.VectorSubcoreMesh(core_axis_name="c", subcore_axis_name="s"),
        scratch_types=scratch,
        compiler_params=pltpu.CompilerParams(
            needs_layout_passes=False, use_tc_tiling_on_sc=False),
    )
    return fn(user_id, gender_id, job_id, user_city_id, age_bucket,
              ulab, ulen, item_id, category_id, item_city_id, ilab, ilen,
              uid_tab, gen_tab, job_tab, city_tab, age_tab,
              iid_tab, cat_tab, lab_tab, pool_w)


def _pack_body(x_ref, o_ref):
    xb = x_ref[...].astype(jnp.bfloat16)
    lo = lax.bitcast_convert_type(xb[:, :NP], jnp.uint16).astype(jnp.uint32)
    hi = lax.bitcast_convert_type(xb[:, NP:], jnp.uint16).astype(jnp.uint32)
    o_ref[...] = lax.bitcast_convert_type(lo | (hi << 16), _i32)


def _pack_label_table(label_table):
    vocab = label_table.shape[0]
    vb = 1000
    return pl.pallas_call(
        _pack_body,
        grid=(vocab // vb,),
        in_specs=[pl.BlockSpec((vb, DLAB), lambda i: (i, 0))],
        out_specs=pl.BlockSpec((vb, NP), lambda i: (i, 0)),
        out_shape=jax.ShapeDtypeStruct((vocab, NP), _i32),
    )(label_table)


BS = 2048
NB = B // BS


def _bmm(a, b):
    return lax.dot(a.astype(jnp.bfloat16), b.astype(jnp.bfloat16),
                   preferred_element_type=_f32)


def _tc_body(uid, gen, job, ucity, age, upool, iid, cat, icity, ipool,
             Wu1, bu1, Wu2, bu2, Wi1, bi1, Wi2, bi2, out_ref):
    w = Wu1[...]
    h = (_bmm(uid[...], w[0:64]) + _bmm(gen[...], w[64:80])
         + _bmm(job[...], w[80:96]) + _bmm(ucity[...], w[96:112])
         + _bmm(age[...], w[112:128]) + _bmm(upool[...], w[128:160])
         + bu1[...])
    h = jnp.maximum(h, 0.0)
    uv = _bmm(h, Wu2[...]) + bu2[...]
    wi = Wi1[...]
    hi = (_bmm(iid[...], wi[0:64]) + _bmm(cat[...], wi[64:96])
          + _bmm(icity[...], wi[96:112]) + _bmm(ipool[...], wi[112:144])
          + bi1[...])
    iv = _bmm(hi, Wi2[...]) + bi2[...]
    s = jnp.sum(uv * iv, axis=1)
    out_ref[0, 0, :] = 1.0 / (1.0 + jnp.exp(-s))


def _tc_mlp(uid, gen, job, ucity, age, upool, iid, cat, icity, ipool,
            Wu1, bu1, Wu2, bu2, Wi1, bi1, Wi2, bi2):
    def row_spec(dim):
        return pl.BlockSpec((BS, dim), lambda i: (i, 0))

    def full_spec(shape):
        return pl.BlockSpec(shape, lambda i: (0, 0))

    out = pl.pallas_call(
        _tc_body,
        grid=(NB,),
        in_specs=[
            row_spec(64), row_spec(16), row_spec(16), row_spec(16),
            row_spec(16), row_spec(32), row_spec(64), row_spec(32),
            row_spec(16), row_spec(32),
            full_spec((160, 256)), full_spec((1, 256)),
            full_spec((256, 128)), full_spec((1, 128)),
            full_spec((144, 256)), full_spec((1, 256)),
            full_spec((256, 128)), full_spec((1, 128)),
        ],
        out_specs=pl.BlockSpec((1, 1, BS), lambda i: (i, 0, 0)),
        out_shape=jax.ShapeDtypeStruct((NB, 1, BS), _f32),
    )(uid, gen, job, ucity, age, upool, iid, cat, icity, ipool,
      Wu1, bu1, Wu2, bu2, Wi1, bi1, Wi2, bi2)
    return out.reshape(B)


def kernel(user_id, gender_id, job_id, user_city_id, age_bucket,
           user_label_list, user_label_length,
           item_id, category_id, item_city_id,
           item_label_list, item_label_length,
           user_id_table, gender_table, job_table, city_table, age_table,
           item_id_table, category_table, label_table, pool_w,
           Wu1, bu1, Wu2, bu2, Wi1, bi1, Wi2, bi2):
    ii = lambda x: x.astype(_i32)
    lab_packed = _pack_label_table(label_table)
    outs = _sc_gather_pool(
        ii(user_id), ii(gender_id), ii(job_id), ii(user_city_id),
        ii(age_bucket), ii(user_label_list), ii(user_label_length),
        ii(item_id), ii(category_id), ii(item_city_id),
        ii(item_label_list), ii(item_label_length),
        user_id_table, gender_table, job_table, city_table, age_table,
        item_id_table, category_table, lab_packed, pool_w)
    (uid_r, gen_r, job_r, ucity_r, age_r, upool,
     iid_r, cat_r, icity_r, ipool) = outs
    return _tc_mlp(uid_r, gen_r, job_r, ucity_r, age_r, upool,
                   iid_r, cat_r, icity_r, ipool,
                   Wu1, bu1.reshape(1, 256), Wu2, bu2.reshape(1, 128),
                   Wi1, bi1.reshape(1, 256), Wi2, bi2.reshape(1, 128))


# speedup vs baseline: 1.0127x; 1.0127x over previous
import jax
import jax.numpy as jnp
from jax import lax
from jax.experimental import pallas as pl
from jax.experimental.pallas import tpu as pltpu
from jax.experimental.pallas import tpu_sc as plsc

B = 16384
L = 50
DLAB = 32
NEG = -1e9

_info = plsc.get_sparse_core_info()
NC = _info.num_cores
NS = _info.num_subcores
NW = NC * NS
EPW = B // NW
CE = 16
NCHUNK = EPW // CE
FE = 128
NFCH = EPW // FE

_f32 = jnp.float32
_i32 = jnp.int32


def _splat_i(v):
    return jnp.full((16,), v, _i32)


NP = DLAB // 2


def _unpack_pair(g):
    return plsc.unpack(plsc.bitcast(g, jnp.bfloat16),
                       format=plsc.PackFormat.INTERLEAVED,
                       preferred_element_type=_f32)


def _pool_compute(emb_ref, scores_ref, len_ref, p, eofs, pool_ref, w_v):
    iota = lax.iota(_i32, 16)
    rowb = iota * L + eofs
    lenv = jnp.maximum(len_ref[pl.ds(p * CE, CE)], 1)
    cols = [(iota + p0) & (NP - 1) for p0 in range(NP)]
    w_e = [plsc.load_gather(w_v, [cols[p0] * 2]) for p0 in range(NP)]
    w_o = [plsc.load_gather(w_v, [cols[p0] * 2 + 1]) for p0 in range(NP)]

    def s_body(l, m):
        row = rowb + l
        a = [jnp.zeros((16,), _f32) for _ in range(4)]
        for p0 in range(NP):
            g = plsc.load_gather(emb_ref, [row, cols[p0]])
            ge, go = _unpack_pair(g)
            a[p0 % 2] = a[p0 % 2] + ge * w_e[p0]
            a[2 + p0 % 2] = a[2 + p0 % 2] + go * w_o[p0]
        acc = (a[0] + a[1]) + (a[2] + a[3])
        s = jnp.where(l < lenv, acc, jnp.full((16,), NEG, _f32))
        scores_ref[pl.ds(l * 16, 16)] = s
        return jnp.maximum(m, s)

    m = lax.fori_loop(0, L, s_body, jnp.full((16,), NEG, _f32))

    def e_body(l, ssum):
        s = scores_ref[pl.ds(l * 16, 16)]
        e = jnp.exp(s - m)
        scores_ref[pl.ds(l * 16, 16)] = e
        return ssum + e

    ssum = lax.fori_loop(0, L, e_body, jnp.zeros((16,), _f32))
    r = 1.0 / ssum

    half_np = NP // 2
    for half in range(2):
        def w_body(l, accs):
            e = scores_ref[pl.ds(l * 16, 16)]
            row = rowb + l
            new = []
            for k in range(half_np):
                p0 = half * half_np + k
                g = plsc.load_gather(emb_ref, [row, cols[p0]])
                ge, go = _unpack_pair(g)
                new.append(accs[2 * k] + e * ge)
                new.append(accs[2 * k + 1] + e * go)
            return tuple(new)

        res = lax.fori_loop(0, L, w_body, tuple(
            jnp.zeros((16,), _f32) for _ in range(NP)))
        for k in range(half_np):
            p0 = half * half_np + k
            plsc.store_scatter(pool_ref, [iota, cols[p0] * 2],
                               res[2 * k] * r)
            plsc.store_scatter(pool_ref, [iota, cols[p0] * 2 + 1],
                               res[2 * k + 1] * r)


def _sc_body(user_id, gender_id, job_id, user_city_id, age_bucket,
             ulab, ulen, item_id, category_id, item_city_id, ilab, ilen,
             uid_tab, gen_tab, job_tab, city_tab, age_tab,
             iid_tab, cat_tab, lab_tab, pool_w,
             uid_o, gen_o, job_o, ucity_o, age_o, upool_o,
             iid_o, cat_o, icity_o, ipool_o,
             uidx_v, iidx_v, uemb_v, iemb_v, scores_v,
             ulen_v, ilen_v, w_v, pool_u_v, pool_i_v,
             fi_uid, fi_gen, fi_job, fi_ucity, fi_age, fi_iid, fi_cat,
             fi_icity,
             fr_uid, fr_gen, fr_job, fr_ucity, fr_age, fr_iid, fr_cat,
             fr_icity,
             sem_a, sem_b0, sem_b1, sem_c):
    wid = lax.axis_index("s") * NC + lax.axis_index("c")
    wbase = pl.multiple_of(wid * EPW, 128)
    pltpu.sync_copy(pool_w, w_v)

    fields = [
        (user_id, uid_tab, fi_uid, fr_uid, uid_o),
        (gender_id, gen_tab, fi_gen, fr_gen, gen_o),
        (job_id, job_tab, fi_job, fr_job, job_o),
        (user_city_id, city_tab, fi_ucity, fr_ucity, ucity_o),
        (age_bucket, age_tab, fi_age, fr_age, age_o),
        (item_id, iid_tab, fi_iid, fr_iid, iid_o),
        (category_id, cat_tab, fi_cat, fr_cat, cat_o),
        (item_city_id, city_tab, fi_icity, fr_icity, icity_o),
    ]
    sem_b = (sem_b0, sem_b1)

    def fire_idx(nbase, p1):
        ds = [pltpu.async_copy(ulab.at[pl.ds(nbase, CE)], uidx_v.at[p1],
                               sem_a),
              pltpu.async_copy(ilab.at[pl.ds(nbase, CE)], iidx_v.at[p1],
                               sem_a),
              pltpu.async_copy(ulen.at[pl.ds(nbase, CE)],
                               ulen_v.at[pl.ds(p1 * CE, CE)], sem_a),
              pltpu.async_copy(ilen.at[pl.ds(nbase, CE)],
                               ilen_v.at[pl.ds(p1 * CE, CE)], sem_a)]
        ds += [pltpu.async_copy(src.at[pl.ds(nbase, CE)], idx_v.at[p1],
                                sem_a)
               for (src, _, idx_v, _, _) in fields]
        for d in ds:
            d.wait()

    def fire_gathers(p1):
        s = sem_b[p1]
        for j in range(CE):
            pltpu.async_copy(lab_tab.at[uidx_v.at[p1, j]],
                             uemb_v.at[pl.ds(p1 * CE * L + j * L, L)], s)
            pltpu.async_copy(lab_tab.at[iidx_v.at[p1, j]],
                             iemb_v.at[pl.ds(p1 * CE * L + j * L, L)], s)
        for (_, tab, idx_v, row_v, _) in fields:
            pltpu.async_copy(tab.at[idx_v.at[p1]], row_v.at[p1], s)

    def wait_gathers(p):
        s = sem_b[p]
        dummy = lab_tab.at[pl.ds(0, CE * L)]
        pltpu.make_async_copy(dummy, uemb_v.at[pl.ds(p * CE * L, CE * L)],
                              s).wait()
        pltpu.make_async_copy(dummy, iemb_v.at[pl.ds(p * CE * L, CE * L)],
                              s).wait()
        for (_, _, _, row_v, out) in fields:
            pltpu.make_async_copy(out.at[pl.ds(0, CE)], row_v.at[p],
                                  s).wait()

    fire_idx(wbase, 0)
    fire_gathers(0)

    def c_body(c, carry):
        even = (c & 1) == 0
        not_last = c < NCHUNK - 1
        nbase = pl.multiple_of(wbase + (c + 1) * CE, 16)
        base = pl.multiple_of(wbase + c * CE, 16)
        p = c & 1

        @pl.when(even & not_last)
        def _():
            fire_idx(nbase, 1)
            fire_gathers(1)

        @pl.when(jnp.logical_not(even) & not_last)
        def _():
            fire_idx(nbase, 0)
            fire_gathers(0)

        @pl.when(even)
        def _():
            wait_gathers(0)

        @pl.when(jnp.logical_not(even))
        def _():
            wait_gathers(1)

        eofs = p * CE * L
        _pool_compute(uemb_v, scores_v, ulen_v, p, eofs, pool_u_v, w_v)
        du = pltpu.async_copy(pool_u_v, upool_o.at[pl.ds(base, CE)], sem_c)
        _pool_compute(iemb_v, scores_v, ilen_v, p, eofs, pool_i_v, w_v)
        di = pltpu.async_copy(pool_i_v, ipool_o.at[pl.ds(base, CE)], sem_c)
        dfs = [pltpu.async_copy(row_v.at[p], out.at[pl.ds(base, CE)], sem_c)
               for (_, _, _, row_v, out) in fields]
        du.wait()
        di.wait()
        for d in dfs:
            d.wait()
        return carry

    lax.fori_loop(0, NCHUNK, c_body, 0)


def _sc_gather_pool(user_id, gender_id, job_id, user_city_id, age_bucket,
                    ulab, ulen, item_id, category_id, item_city_id,
                    ilab, ilen, uid_tab, gen_tab, job_tab, city_tab,
                    age_tab, iid_tab, cat_tab, lab_tab, pool_w):
    out_type = [
        jax.ShapeDtypeStruct((B, 64), _f32),
        jax.ShapeDtypeStruct((B, 16), _f32),
        jax.ShapeDtypeStruct((B, 16), _f32),
        jax.ShapeDtypeStruct((B, 16), _f32),
        jax.ShapeDtypeStruct((B, 16), _f32),
        jax.ShapeDtypeStruct((B, 32), _f32),
        jax.ShapeDtypeStruct((B, 64), _f32),
        jax.ShapeDtypeStruct((B, 32), _f32),
        jax.ShapeDtypeStruct((B, 16), _f32),
        jax.ShapeDtypeStruct((B, 32), _f32),
    ]
    scratch = [
        pltpu.VMEM((2, CE, L), _i32),
        pltpu.VMEM((2, CE, L), _i32),
        pltpu.VMEM((2 * CE * L, NP), _i32),
        pltpu.VMEM((2 * CE * L, NP), _i32),
        pltpu.VMEM((L * 16,), _f32),
        pltpu.VMEM((2 * CE,), _i32),
        pltpu.VMEM((2 * CE,), _i32),
        pltpu.VMEM((DLAB,), _f32),
        pltpu.VMEM((CE, DLAB), _f32),
        pltpu.VMEM((CE, DLAB), _f32),
        pltpu.VMEM((2, CE), _i32),
        pltpu.VMEM((2, CE), _i32),
        pltpu.VMEM((2, CE), _i32),
        pltpu.VMEM((2, CE), _i32),
        pltpu.VMEM((2, CE), _i32),
        pltpu.VMEM((2, CE), _i32),
        pltpu.VMEM((2, CE), _i32),
        pltpu.VMEM((2, CE), _i32),
        pltpu.VMEM((2, CE, 64), _f32),
        pltpu.VMEM((2, CE, 16), _f32),
        pltpu.VMEM((2, CE, 16), _f32),
        pltpu.VMEM((2, CE, 16), _f32),
        pltpu.VMEM((2, CE, 16), _f32),
        pltpu.VMEM((2, CE, 64), _f32),
        pltpu.VMEM((2, CE, 32), _f32),
        pltpu.VMEM((2, CE, 16), _f32),
        pltpu.SemaphoreType.DMA,
        pltpu.SemaphoreType.DMA,
        pltpu.SemaphoreType.DMA,
        pltpu.SemaphoreType.DMA,
    ]
    fn = pl.kernel(
        _sc_body,
        out_type=out_type,
        mesh=plsc.VectorSubcoreMesh(core_axis_name="c", subcore_axis_name="s"),
        scratch_types=scratch,
        compiler_params=pltpu.CompilerParams(
            needs_layout_passes=False, use_tc_tiling_on_sc=False),
    )
    return fn(user_id, gender_id, job_id, user_city_id, age_bucket,
              ulab, ulen, item_id, category_id, item_city_id, ilab, ilen,
              uid_tab, gen_tab, job_tab, city_tab, age_tab,
              iid_tab, cat_tab, lab_tab, pool_w)


BS = 2048
NB = B // BS


def _bmm(a, b):
    return lax.dot(a.astype(jnp.bfloat16), b.astype(jnp.bfloat16),
                   preferred_element_type=_f32)


def _tc_body(uid, gen, job, ucity, age, upool, iid, cat, icity, ipool,
             Wu1, bu1, Wu2, bu2, Wi1, bi1, Wi2, bi2, out_ref):
    w = Wu1[...]
    h = (_bmm(uid[...], w[0:64]) + _bmm(gen[...], w[64:80])
         + _bmm(job[...], w[80:96]) + _bmm(ucity[...], w[96:112])
         + _bmm(age[...], w[112:128]) + _bmm(upool[...], w[128:160])
         + bu1[...])
    h = jnp.maximum(h, 0.0)
    uv = _bmm(h, Wu2[...]) + bu2[...]
    wi = Wi1[...]
    hi = (_bmm(iid[...], wi[0:64]) + _bmm(cat[...], wi[64:96])
          + _bmm(icity[...], wi[96:112]) + _bmm(ipool[...], wi[112:144])
          + bi1[...])
    iv = _bmm(hi, Wi2[...]) + bi2[...]
    s = jnp.sum(uv * iv, axis=1)
    out_ref[0, 0, :] = 1.0 / (1.0 + jnp.exp(-s))


def _tc_mlp(uid, gen, job, ucity, age, upool, iid, cat, icity, ipool,
            Wu1, bu1, Wu2, bu2, Wi1, bi1, Wi2, bi2):
    def row_spec(dim):
        return pl.BlockSpec((BS, dim), lambda i: (i, 0))

    def full_spec(shape):
        return pl.BlockSpec(shape, lambda i: (0, 0))

    out = pl.pallas_call(
        _tc_body,
        grid=(NB,),
        in_specs=[
            row_spec(64), row_spec(16), row_spec(16), row_spec(16),
            row_spec(16), row_spec(32), row_spec(64), row_spec(32),
            row_spec(16), row_spec(32),
            full_spec((160, 256)), full_spec((1, 256)),
            full_spec((256, 128)), full_spec((1, 128)),
            full_spec((144, 256)), full_spec((1, 256)),
            full_spec((256, 128)), full_spec((1, 128)),
        ],
        out_specs=pl.BlockSpec((1, 1, BS), lambda i: (i, 0, 0)),
        out_shape=jax.ShapeDtypeStruct((NB, 1, BS), _f32),
    )(uid, gen, job, ucity, age, upool, iid, cat, icity, ipool,
      Wu1, bu1, Wu2, bu2, Wi1, bi1, Wi2, bi2)
    return out.reshape(B)


def kernel(user_id, gender_id, job_id, user_city_id, age_bucket,
           user_label_list, user_label_length,
           item_id, category_id, item_city_id,
           item_label_list, item_label_length,
           user_id_table, gender_table, job_table, city_table, age_table,
           item_id_table, category_table, label_table, pool_w,
           Wu1, bu1, Wu2, bu2, Wi1, bi1, Wi2, bi2):
    ii = lambda x: x.astype(_i32)
    vocab = label_table.shape[0]
    lab_packed = lax.bitcast_convert_type(
        label_table.astype(jnp.bfloat16).reshape(vocab, NP, 2), _i32)
    outs = _sc_gather_pool(
        ii(user_id), ii(gender_id), ii(job_id), ii(user_city_id),
        ii(age_bucket), ii(user_label_list), ii(user_label_length),
        ii(item_id), ii(category_id), ii(item_city_id),
        ii(item_label_list), ii(item_label_length),
        user_id_table, gender_table, job_table, city_table, age_table,
        item_id_table, category_table, lab_packed, pool_w)
    (uid_r, gen_r, job_r, ucity_r, age_r, upool,
     iid_r, cat_r, icity_r, ipool) = outs
    return _tc_mlp(uid_r, gen_r, job_r, ucity_r, age_r, upool,
                   iid_r, cat_r, icity_r, ipool,
                   Wu1, bu1.reshape(1, 256), Wu2, bu2.reshape(1, 128),
                   Wi1, bi1.reshape(1, 256), Wi2, bi2.reshape(1, 128))
